# Initial kernel scaffold; baseline (speedup 1.0000x reference)
import functools

import jax
import jax.numpy as jnp
from jax import lax
from jax.experimental import pallas as pl
from jax.experimental.pallas import tpu as pltpu
from jax.experimental.pallas import tpu_sc as plsc

G = 25600
N = 163840
K = 16
C = 14
RW = 32
NW = 32
PTS = N // NW
P = 16
ROWS = P * K
NB = PTS // P


def _prep_body(m_ref, s_ref, r_ref, o_ref, sem_ref, out_ref):
    w = r_ref[0:1, :]
    x = r_ref[1:2, :]
    y = r_ref[2:3, :]
    z = r_ref[3:4, :]
    n2 = w * w + x * x + y * y + z * z
    t = 2.0 / n2
    r00 = 1.0 - t * (y * y + z * z)
    r01 = t * (x * y - w * z)
    r02 = t * (x * z + w * y)
    r10 = t * (x * y + w * z)
    r11 = 1.0 - t * (x * x + z * z)
    r12 = t * (y * z - w * x)
    r20 = t * (x * z - w * y)
    r21 = t * (y * z + w * x)
    r22 = 1.0 - t * (x * x + y * y)
    s0 = s_ref[0:1, :]
    s1 = s_ref[1:2, :]
    s2 = s_ref[2:3, :]
    a0 = 1.0 / (s0 * s0 + 1e-6)
    a1 = 1.0 / (s1 * s1 + 1e-6)
    a2 = 1.0 / (s2 * s2 + 1e-6)
    c00 = a0 * r00 * r00 + a1 * r10 * r10 + a2 * r20 * r20
    c11 = a0 * r01 * r01 + a1 * r11 * r11 + a2 * r21 * r21
    c22 = a0 * r02 * r02 + a1 * r12 * r12 + a2 * r22 * r22
    c01 = a0 * r00 * r01 + a1 * r10 * r11 + a2 * r20 * r21
    c02 = a0 * r00 * r02 + a1 * r10 * r12 + a2 * r20 * r22
    c12 = a0 * r01 * r02 + a1 * r11 * r12 + a2 * r21 * r22
    out_ref[0:3, :] = m_ref[...]
    out_ref[3:4, :] = c00
    out_ref[4:5, :] = c11
    out_ref[5:6, :] = c22
    out_ref[6:7, :] = c01
    out_ref[7:8, :] = c02
    out_ref[8:9, :] = c12
    out_ref[9:10, :] = o_ref[...]
    out_ref[10:24, :] = sem_ref[...]
    out_ref[24:32, :] = jnp.zeros((8, out_ref.shape[1]), jnp.float32)


def _make_table(means, scales, rotations, opacities, semantics):
    tableT = pl.pallas_call(
        _prep_body,
        out_shape=jax.ShapeDtypeStruct((RW, G), jnp.float32),
    )(means.T, scales.T, rotations.T, opacities.T, semantics.T)
    return tableT.T


_MESH = plsc.VectorSubcoreMesh(core_axis_name="c", subcore_axis_name="s")


@functools.partial(
    pl.kernel,
    mesh=_MESH,
    out_type=[
        jax.ShapeDtypeStruct((C, N), jnp.float32),
        jax.ShapeDtypeStruct((N,), jnp.int32),
    ],
    scratch_types=[
        pltpu.VMEM((PTS,), jnp.float32),
        pltpu.VMEM((PTS,), jnp.float32),
        pltpu.VMEM((PTS,), jnp.float32),
        pltpu.VMEM((ROWS,), jnp.int32),
        pltpu.VMEM((ROWS, RW), jnp.float32),
        pltpu.VMEM((C, PTS), jnp.float32),
        pltpu.VMEM((PTS,), jnp.int32),
        pltpu.SemaphoreType.DMA,
    ],
)
def _agg_kernel(table, xs, ys, zs, nbr, aggT, occ,
                xs_v, ys_v, zs_v, idx_v, rows_v, agg_v, occ_v, sem):
    wid = lax.axis_index("s") * 2 + lax.axis_index("c")
    base = wid * PTS
    pltpu.sync_copy(xs.at[pl.ds(base, PTS)], xs_v)
    pltpu.sync_copy(ys.at[pl.ds(base, PTS)], ys_v)
    pltpu.sync_copy(zs.at[pl.ds(base, PTS)], zs_v)
    iota = lax.iota(jnp.int32, 16)

    def body(b, carry):
        pbase = b * P
        goff = (base + pbase) * K
        pltpu.sync_copy(nbr.at[pl.ds(goff, ROWS)], idx_v)
        pltpu.async_copy(table.at[idx_v], rows_v, sem).wait()
        x = xs_v[pl.ds(pbase, P)]
        y = ys_v[pl.ds(pbase, P)]
        z = zs_v[pl.ds(pbase, P)]
        acc = [jnp.zeros((16,), jnp.float32) for _ in range(C)]
        for k in range(K):
            rid = iota * K + k

            def g(f, rid=rid):
                return plsc.load_gather(
                    rows_v, [rid, jnp.full((16,), f, jnp.int32)])

            dx = x - g(0)
            dy = y - g(1)
            dz = z - g(2)
            maha = (g(3) * dx * dx + g(4) * dy * dy + g(5) * dz * dz
                    + 2.0 * (g(6) * dx * dy + g(7) * dx * dz + g(8) * dy * dz))
            wgt = g(9) * jnp.exp(-0.5 * maha)
            for c in range(C):
                acc[c] = acc[c] + wgt * g(10 + c)
        best = acc[0]
        bi = jnp.zeros((16,), jnp.int32)
        for c in range(1, C):
            m = acc[c] > best
            best = jnp.where(m, acc[c], best)
            bi = jnp.where(m, jnp.full((16,), c, jnp.int32), bi)
        for c in range(C):
            agg_v[c, pl.ds(pbase, P)] = acc[c]
        occ_v[pl.ds(pbase, P)] = bi
        return carry

    lax.fori_loop(0, NB, body, 0)
    for c in range(C):
        pltpu.sync_copy(agg_v.at[c], aggT.at[c, pl.ds(base, PTS)])
    pltpu.sync_copy(occ_v, occ.at[pl.ds(base, PTS)])


def kernel(means, scales, rotations, opacities, semantics, sampled_xyz,
           neighbor_idx):
    table = _make_table(means, scales, rotations, opacities, semantics)
    xs = sampled_xyz[:, 0]
    ys = sampled_xyz[:, 1]
    zs = sampled_xyz[:, 2]
    nbrflat = neighbor_idx.reshape(-1)
    aggT, occ = _agg_kernel(table, xs, ys, zs, nbrflat)
    return aggT[None], occ[None]


# speedup vs baseline: 4.4196x; 4.4196x over previous
import functools

import jax
import jax.numpy as jnp
from jax import lax
from jax.experimental import pallas as pl
from jax.experimental.pallas import tpu as pltpu
from jax.experimental.pallas import tpu_sc as plsc

G = 25600
N = 163840
K = 16
C = 14
RW = 32
NW = 32
PTS = N // NW
P = 16
ROWS = P * K
NB = PTS // P


def _prep_body(m_ref, c_ref, o_ref, sem_ref, out_ref):
    out_ref[0:3, :] = m_ref[...]
    out_ref[3:12, :] = c_ref[...]
    out_ref[12:13, :] = o_ref[...]
    out_ref[13:27, :] = sem_ref[...]
    out_ref[27:32, :] = jnp.zeros((5, out_ref.shape[1]), jnp.float32)


def _quat_rot(q):
    q = q / jnp.linalg.norm(q, axis=-1, keepdims=True)
    w, x, y, z = q[..., 0], q[..., 1], q[..., 2], q[..., 3]
    r00 = 1 - 2 * (y * y + z * z); r01 = 2 * (x * y - w * z); r02 = 2 * (x * z + w * y)
    r10 = 2 * (x * y + w * z); r11 = 1 - 2 * (x * x + z * z); r12 = 2 * (y * z - w * x)
    r20 = 2 * (x * z - w * y); r21 = 2 * (y * z + w * x); r22 = 1 - 2 * (x * x + y * y)
    row0 = jnp.stack([r00, r01, r02], axis=-1)
    row1 = jnp.stack([r10, r11, r12], axis=-1)
    row2 = jnp.stack([r20, r21, r22], axis=-1)
    return jnp.stack([row0, row1, row2], axis=-2)


def _make_table(means, scales, rotations, opacities, semantics):
    R = _quat_rot(rotations)
    Mb = scales.astype(jnp.bfloat16)[:, :, None] * R.astype(jnp.bfloat16)
    Mf = Mb.astype(jnp.float32)
    p = Mf[:, :, :, None] * Mf[:, :, None, :]
    Cov = (p[:, 0] + p[:, 1]) + p[:, 2]
    Cov = Cov + 1e-6 * jnp.eye(3, dtype=means.dtype)
    CI = jnp.linalg.inv(Cov)
    ui = jax.lax.bitcast_convert_type(CI, jnp.int32)
    ui = (ui + 0x7FFF + ((ui >> 16) & 1)) & jnp.int32(-65536)
    CIb = jax.lax.bitcast_convert_type(ui, jnp.float32)
    cT = jnp.stack([CIb[:, i, j] for i in range(3) for j in range(3)], axis=0)
    tableT = pl.pallas_call(
        _prep_body,
        out_shape=jax.ShapeDtypeStruct((RW, G), jnp.float32),
    )(means.T, cT, opacities.T, semantics.T)
    return tableT.T


@functools.cache
def _build_agg_kernel():
    mesh = plsc.VectorSubcoreMesh(core_axis_name="c", subcore_axis_name="s")
    return pl.kernel(
        _agg_body,
        mesh=mesh,
        compiler_params=pltpu.CompilerParams(
            needs_layout_passes=False, use_tc_tiling_on_sc=False),
        out_type=[
            jax.ShapeDtypeStruct((C, N), jnp.float32),
            jax.ShapeDtypeStruct((N,), jnp.int32),
        ],
        scratch_types=[
            pltpu.VMEM((PTS,), jnp.float32),
            pltpu.VMEM((PTS,), jnp.float32),
            pltpu.VMEM((PTS,), jnp.float32),
            pltpu.VMEM((ROWS,), jnp.int32),
            pltpu.VMEM((ROWS, RW), jnp.float32),
            pltpu.VMEM((C, PTS), jnp.float32),
            pltpu.VMEM((PTS,), jnp.int32),
            pltpu.SemaphoreType.DMA,
        ],
    )


def _agg_body(table, xs, ys, zs, nbr, aggT, occ,
              xs_v, ys_v, zs_v, idx_v, rows_v, agg_v, occ_v, sem):
    wid = lax.axis_index("s") * 2 + lax.axis_index("c")
    base = wid * PTS
    pltpu.sync_copy(xs.at[pl.ds(base, PTS)], xs_v)
    pltpu.sync_copy(ys.at[pl.ds(base, PTS)], ys_v)
    pltpu.sync_copy(zs.at[pl.ds(base, PTS)], zs_v)
    iota = lax.iota(jnp.int32, 16)

    def body(b, carry):
        pbase = b * P
        goff = (base + pbase) * K
        pltpu.sync_copy(nbr.at[pl.ds(goff, ROWS)], idx_v)
        pltpu.async_copy(table.at[idx_v], rows_v, sem).wait()
        x = xs_v[pl.ds(pbase, P)]
        y = ys_v[pl.ds(pbase, P)]
        z = zs_v[pl.ds(pbase, P)]
        acc = [jnp.zeros((16,), jnp.float32) for _ in range(C)]
        def bf16r(v):
            u = plsc.bitcast(v, jnp.int32)
            r = (u + 0x7FFF + ((u >> 16) & 1)) & jnp.int32(-65536)
            return plsc.bitcast(r, jnp.float32)

        for k in range(K):
            rid = iota * K + k

            def g(f, rid=rid):
                return plsc.load_gather(
                    rows_v, [rid, jnp.full((16,), f, jnp.int32)])

            dx = x - g(0)
            dy = y - g(1)
            dz = z - g(2)
            bx, by, bz = bf16r(dx), bf16r(dy), bf16r(dz)
            u0 = (bx * g(3) + by * g(6)) + bz * g(9)
            u1 = (bx * g(4) + by * g(7)) + bz * g(10)
            u2 = (bx * g(5) + by * g(8)) + bz * g(11)
            maha = (u0 * dx + u1 * dy) + u2 * dz
            arg = -0.5 * maha
            e = jnp.where(arg < -87.33654, 0.0, jnp.exp(arg))
            wgt = g(12) * e
            wgt = jnp.where(wgt < 1.1754944e-38, 0.0, wgt)
            for c in range(C):
                acc[c] = acc[c] + wgt * g(13 + c)
        best = acc[0]
        bi = jnp.zeros((16,), jnp.int32)
        for c in range(1, C):
            m = acc[c] > best
            best = jnp.where(m, acc[c], best)
            bi = jnp.where(m, jnp.full((16,), c, jnp.int32), bi)
        for c in range(C):
            agg_v[c, pl.ds(pbase, P)] = acc[c]
        occ_v[pl.ds(pbase, P)] = bi
        return carry

    lax.fori_loop(0, NB, body, 0)
    for c in range(C):
        pltpu.sync_copy(agg_v.at[c], aggT.at[c, pl.ds(base, PTS)])
    pltpu.sync_copy(occ_v, occ.at[pl.ds(base, PTS)])


def kernel(means, scales, rotations, opacities, semantics, sampled_xyz,
           neighbor_idx):
    table = _make_table(means, scales, rotations, opacities, semantics)
    xs = sampled_xyz[:, 0]
    ys = sampled_xyz[:, 1]
    zs = sampled_xyz[:, 2]
    nbrflat = neighbor_idx.reshape(-1)
    aggT, occ = _build_agg_kernel()(table, xs, ys, zs, nbrflat)
    return aggT[None], occ[None]


# speedup vs baseline: 4.4703x; 1.0115x over previous
import functools

import jax
import jax.numpy as jnp
from jax import lax
from jax.experimental import pallas as pl
from jax.experimental.pallas import tpu as pltpu
from jax.experimental.pallas import tpu_sc as plsc

G = 25600
N = 163840
K = 16
C = 14
RW = 32
NW = 32
PTS = N // NW
P = 16
ROWS = P * K
NB = PTS // P


def _prep_body(m_ref, c_ref, o_ref, sem_ref, out_ref):
    out_ref[0:3, :] = m_ref[...]
    out_ref[3:12, :] = c_ref[...]
    out_ref[12:13, :] = o_ref[...]
    out_ref[13:27, :] = sem_ref[...]
    out_ref[27:32, :] = jnp.zeros((5, out_ref.shape[1]), jnp.float32)


def _quat_rot(q):
    q = q / jnp.linalg.norm(q, axis=-1, keepdims=True)
    w, x, y, z = q[..., 0], q[..., 1], q[..., 2], q[..., 3]
    r00 = 1 - 2 * (y * y + z * z); r01 = 2 * (x * y - w * z); r02 = 2 * (x * z + w * y)
    r10 = 2 * (x * y + w * z); r11 = 1 - 2 * (x * x + z * z); r12 = 2 * (y * z - w * x)
    r20 = 2 * (x * z - w * y); r21 = 2 * (y * z + w * x); r22 = 1 - 2 * (x * x + y * y)
    row0 = jnp.stack([r00, r01, r02], axis=-1)
    row1 = jnp.stack([r10, r11, r12], axis=-1)
    row2 = jnp.stack([r20, r21, r22], axis=-1)
    return jnp.stack([row0, row1, row2], axis=-2)


def _make_table(means, scales, rotations, opacities, semantics):
    R = _quat_rot(rotations)
    Mb = scales.astype(jnp.bfloat16)[:, :, None] * R.astype(jnp.bfloat16)
    Mf = Mb.astype(jnp.float32)
    p = Mf[:, :, :, None] * Mf[:, :, None, :]
    Cov = (p[:, 0] + p[:, 1]) + p[:, 2]
    Cov = Cov + 1e-6 * jnp.eye(3, dtype=means.dtype)
    CI = jnp.linalg.inv(Cov)
    ui = jax.lax.bitcast_convert_type(CI, jnp.int32)
    ui = (ui + 0x7FFF + ((ui >> 16) & 1)) & jnp.int32(-65536)
    CIb = jax.lax.bitcast_convert_type(ui, jnp.float32)
    cT = jnp.stack([CIb[:, i, j] for i in range(3) for j in range(3)], axis=0)
    tableT = pl.pallas_call(
        _prep_body,
        out_shape=jax.ShapeDtypeStruct((RW, G), jnp.float32),
    )(means.T, cT, opacities.T, semantics.T)
    return tableT.T


@functools.cache
def _build_agg_kernel():
    mesh = plsc.VectorSubcoreMesh(core_axis_name="c", subcore_axis_name="s")
    return pl.kernel(
        _agg_body,
        mesh=mesh,
        compiler_params=pltpu.CompilerParams(
            needs_layout_passes=False, use_tc_tiling_on_sc=False),
        out_type=[
            jax.ShapeDtypeStruct((C, N), jnp.float32),
            jax.ShapeDtypeStruct((N,), jnp.int32),
        ],
        scratch_types=[
            pltpu.VMEM((PTS,), jnp.float32),
            pltpu.VMEM((PTS,), jnp.float32),
            pltpu.VMEM((PTS,), jnp.float32),
            pltpu.VMEM((ROWS,), jnp.int32),
            pltpu.VMEM((ROWS,), jnp.int32),
            pltpu.VMEM((ROWS, RW), jnp.float32),
            pltpu.VMEM((ROWS, RW), jnp.float32),
            pltpu.VMEM((C, PTS), jnp.float32),
            pltpu.VMEM((PTS,), jnp.int32),
            pltpu.SemaphoreType.DMA,
            pltpu.SemaphoreType.DMA,
        ],
    )


def _agg_body(table, xs, ys, zs, nbr, aggT, occ,
              xs_v, ys_v, zs_v, idx0_v, idx1_v, rows0_v, rows1_v,
              agg_v, occ_v, sem0, sem1):
    wid = lax.axis_index("s") * 2 + lax.axis_index("c")
    base = wid * PTS
    pltpu.sync_copy(xs.at[pl.ds(base, PTS)], xs_v)
    pltpu.sync_copy(ys.at[pl.ds(base, PTS)], ys_v)
    pltpu.sync_copy(zs.at[pl.ds(base, PTS)], zs_v)
    iota = lax.iota(jnp.int32, 16)

    def start(b, idx_v, rows_v, sem):
        goff = (base + b * P) * K
        pltpu.sync_copy(nbr.at[pl.ds(goff, ROWS)], idx_v)
        pltpu.async_copy(table.at[idx_v], rows_v, sem)

    def drain(idx_v, rows_v, sem):
        pltpu.make_async_copy(table.at[idx_v], rows_v, sem).wait()

    def compute(b, rows_v):
        pbase = b * P
        x = xs_v[pl.ds(pbase, P)]
        y = ys_v[pl.ds(pbase, P)]
        z = zs_v[pl.ds(pbase, P)]
        acc = [jnp.zeros((16,), jnp.float32) for _ in range(C)]
        def bf16r(v):
            u = plsc.bitcast(v, jnp.int32)
            r = (u + 0x7FFF + ((u >> 16) & 1)) & jnp.int32(-65536)
            return plsc.bitcast(r, jnp.float32)

        for k in range(K):
            rid = iota * K + k

            def g(f, rid=rid):
                return plsc.load_gather(
                    rows_v, [rid, jnp.full((16,), f, jnp.int32)])

            dx = x - g(0)
            dy = y - g(1)
            dz = z - g(2)
            bx, by, bz = bf16r(dx), bf16r(dy), bf16r(dz)
            u0 = (bx * g(3) + by * g(6)) + bz * g(9)
            u1 = (bx * g(4) + by * g(7)) + bz * g(10)
            u2 = (bx * g(5) + by * g(8)) + bz * g(11)
            maha = (u0 * dx + u1 * dy) + u2 * dz
            arg = -0.5 * maha
            e = jnp.where(arg < -87.33654, 0.0, jnp.exp(arg))
            wgt = g(12) * e
            wgt = jnp.where(wgt < 1.1754944e-38, 0.0, wgt)
            for c in range(C):
                acc[c] = acc[c] + wgt * g(13 + c)
        best = acc[0]
        bi = jnp.zeros((16,), jnp.int32)
        for c in range(1, C):
            m = acc[c] > best
            best = jnp.where(m, acc[c], best)
            bi = jnp.where(m, jnp.full((16,), c, jnp.int32), bi)
        for c in range(C):
            agg_v[c, pl.ds(pbase, P)] = acc[c]
        occ_v[pl.ds(pbase, P)] = bi

    start(0, idx0_v, rows0_v, sem0)

    def body(i, carry):
        b0 = 2 * i
        start(b0 + 1, idx1_v, rows1_v, sem1)
        drain(idx0_v, rows0_v, sem0)
        compute(b0, rows0_v)
        start(jnp.minimum(b0 + 2, NB - 1), idx0_v, rows0_v, sem0)
        drain(idx1_v, rows1_v, sem1)
        compute(b0 + 1, rows1_v)
        return carry

    lax.fori_loop(0, NB // 2, body, 0)
    drain(idx0_v, rows0_v, sem0)
    for c in range(C):
        pltpu.sync_copy(agg_v.at[c], aggT.at[c, pl.ds(base, PTS)])
    pltpu.sync_copy(occ_v, occ.at[pl.ds(base, PTS)])


def kernel(means, scales, rotations, opacities, semantics, sampled_xyz,
           neighbor_idx):
    table = _make_table(means, scales, rotations, opacities, semantics)
    xs = sampled_xyz[:, 0]
    ys = sampled_xyz[:, 1]
    zs = sampled_xyz[:, 2]
    nbrflat = neighbor_idx.reshape(-1)
    aggT, occ = _build_agg_kernel()(table, xs, ys, zs, nbrflat)
    return aggT[None], occ[None]
